# baseline (speedup 1.0000x reference)
import functools

import jax
import jax.numpy as jnp
from jax import lax
from jax.experimental import pallas as pl
from jax.experimental.pallas import tpu as pltpu
from jax.experimental.pallas import tpu_sc as plsc


def _make_transpose(dim: int, vocab: int):
    assert dim == 32
    vb = 800
    n_blocks = vocab // vb
    assert vocab % vb == 0

    info = plsc.get_sparse_core_info()
    nc, ns = info.num_cores, info.num_subcores
    nw = nc * ns

    mesh = plsc.VectorSubcoreMesh(core_axis_name="c", subcore_axis_name="s")

    @functools.partial(
        pl.kernel,
        mesh=mesh,
        out_type=jax.ShapeDtypeStruct((vocab, dim), jnp.float32),
        scratch_types=[
            pltpu.VMEM((dim * vb,), jnp.float32),
            pltpu.VMEM((vb, dim), jnp.float32),
            pltpu.SemaphoreType.DMA,
        ],
        compiler_params=pltpu.CompilerParams(
            use_tc_tiling_on_sc=False, needs_layout_passes=False
        ),
    )
    def transpose_kernel(wt_hbm, out_hbm, blk, obl, sem):
        wid = lax.axis_index("s") * nc + lax.axis_index("c")
        iota = lax.iota(jnp.int32, 16)
        b_lo = iota * vb
        b_hi = (iota + 16) * vb
        rem_w = n_blocks % nw
        n_t = jnp.where(wid < rem_w, n_blocks // nw + 1, n_blocks // nw)

        def body(t, carry):
            v0 = (wid + nw * t) * vb
            copies = [
                pltpu.async_copy(
                    wt_hbm.at[d, pl.ds(v0, vb)],
                    blk.at[pl.ds(d * vb, vb)],
                    sem,
                )
                for d in range(dim)
            ]
            for cp in copies:
                cp.wait()
            for v in range(vb):
                obl[v, pl.ds(0, 16)] = plsc.load_gather(blk, [b_lo + v])
                obl[v, pl.ds(16, 16)] = plsc.load_gather(blk, [b_hi + v])
            pltpu.sync_copy(obl, out_hbm.at[pl.ds(v0, vb)])
            return carry

        lax.fori_loop(0, n_t, body, 0)

    return transpose_kernel


def _make_gather(batch: int, hist: int, vocab: int, dim: int):
    n_total = batch * hist
    info = plsc.get_sparse_core_info()
    nc, ns = info.num_cores, info.num_subcores
    nw = nc * ns
    per_w = n_total // nw
    assert n_total % nw == 0
    chunk = 1600
    assert per_w % chunk == 0
    n_chunks = per_w // chunk

    mesh = plsc.VectorSubcoreMesh(core_axis_name="c", subcore_axis_name="s")

    @functools.partial(
        pl.kernel,
        mesh=mesh,
        out_type=jax.ShapeDtypeStruct((batch, hist, dim), jnp.float32),
        scratch_types=[
            pltpu.VMEM((per_w,), jnp.int32),
            pltpu.VMEM((2, chunk, dim), jnp.float32),
            pltpu.SemaphoreType.DMA,
            pltpu.SemaphoreType.DMA,
            pltpu.SemaphoreType.DMA,
            pltpu.SemaphoreType.DMA,
        ],
        compiler_params=pltpu.CompilerParams(use_tc_tiling_on_sc=False),
    )
    def gather_kernel(idx_hbm, tab_hbm, out_hbm, idx_v, rows_v, g0, g1, o0, o1):
        wid = lax.axis_index("s") * nc + lax.axis_index("c")
        base = wid * per_w
        gsem, osem = [g0, g1], [o0, o1]
        pltpu.sync_copy(idx_hbm.at[pl.ds(base, per_w)], idx_v)

        def start_gather(g):
            b = g % 2
            return pltpu.async_copy(
                tab_hbm.at[idx_v.at[pl.ds(g * chunk, chunk)]],
                rows_v.at[b],
                gsem[b],
            )

        gathers = {0: start_gather(0)}
        stores = [None, None]
        for g in range(n_chunks):
            b = g % 2
            if g + 1 < n_chunks:
                nb = (g + 1) % 2
                if stores[nb] is not None:
                    for s in stores[nb]:
                        s.wait()
                    stores[nb] = None
                gathers[g + 1] = start_gather(g + 1)
            gathers[g].wait()
            row0 = (base + g * chunk) // hist
            stores[b] = [
                pltpu.async_copy(
                    rows_v.at[b, pl.ds(j * hist, hist)],
                    out_hbm.at[row0 + j],
                    osem[b],
                )
                for j in range(chunk // hist)
            ]
        for ss in stores:
            if ss is not None:
                for s in ss:
                    s.wait()

    return gather_kernel


def kernel(token_ids, weight):
    b, h = token_ids.shape
    v, d = weight.shape
    idx = token_ids.reshape(b * h).astype(jnp.int32)
    tab = _make_transpose(d, v)(weight.T)
    return _make_gather(b, h, v, d)(idx, tab)


# speedup vs baseline: 3.8546x; 3.8546x over previous
import functools

import jax
import jax.numpy as jnp
from jax import lax
from jax.experimental import pallas as pl
from jax.experimental.pallas import tpu as pltpu
from jax.experimental.pallas import tpu_sc as plsc


def _make_gather(batch: int, hist: int, vocab: int, dim: int):
    n_total = batch * hist
    info = plsc.get_sparse_core_info()
    nc, ns = info.num_cores, info.num_subcores
    nw = nc * ns
    per_w = n_total // nw
    assert n_total % nw == 0
    chunk = 1600
    assert per_w % chunk == 0
    assert chunk % hist == 0
    n_chunks = per_w // chunk

    mesh = plsc.VectorSubcoreMesh(core_axis_name="c", subcore_axis_name="s")

    @functools.partial(
        pl.kernel,
        mesh=mesh,
        out_type=jax.ShapeDtypeStruct((batch, hist, dim), jnp.float32),
        scratch_types=[
            pltpu.VMEM((per_w,), jnp.int32),
            pltpu.VMEM((2, chunk, dim), jnp.float32),
            pltpu.SemaphoreType.DMA,
            pltpu.SemaphoreType.DMA,
            pltpu.SemaphoreType.DMA,
            pltpu.SemaphoreType.DMA,
        ],
        compiler_params=pltpu.CompilerParams(use_tc_tiling_on_sc=False),
    )
    def gather_kernel(idx_hbm, tab_hbm, out_hbm, idx_v, rows_v, g0, g1, o0, o1):
        wid = lax.axis_index("s") * nc + lax.axis_index("c")
        base = wid * per_w
        gsem, osem = [g0, g1], [o0, o1]
        pltpu.sync_copy(idx_hbm.at[pl.ds(base, per_w)], idx_v)

        def start_gather(g):
            b = g % 2
            return pltpu.async_copy(
                tab_hbm.at[idx_v.at[pl.ds(g * chunk, chunk)]],
                rows_v.at[b],
                gsem[b],
            )

        gathers = {0: start_gather(0)}
        stores = [None, None]
        for g in range(n_chunks):
            b = g % 2
            if g + 1 < n_chunks:
                nb = (g + 1) % 2
                if stores[nb] is not None:
                    for s in stores[nb]:
                        s.wait()
                    stores[nb] = None
                gathers[g + 1] = start_gather(g + 1)
            gathers[g].wait()
            row0 = (base + g * chunk) // hist
            stores[b] = [
                pltpu.async_copy(
                    rows_v.at[b, pl.ds(j * hist, hist)],
                    out_hbm.at[row0 + j],
                    osem[b],
                )
                for j in range(chunk // hist)
            ]
        for ss in stores:
            if ss is not None:
                for s in ss:
                    s.wait()

    return gather_kernel


def kernel(token_ids, weight):
    b, h = token_ids.shape
    v, d = weight.shape
    idx = token_ids.reshape(b * h).astype(jnp.int32)
    return _make_gather(b, h, v, d)(idx, weight)
